# baseline (speedup 1.0000x reference)
import functools

import jax
import jax.numpy as jnp
from jax import lax
from jax.experimental import pallas as pl
from jax.experimental.pallas import tpu as pltpu
from jax.experimental.pallas import tpu_sc as plsc


def _tc_argmin_body(x_ref, w_ref, o_ref):
    xb = x_ref[0]
    w = w_ref[...]
    e2 = jnp.sum(w * w, axis=0)[:, None]
    mm = lax.dot_general(w * -2.0, xb, (((0,), (0,)), ((), ())),
                         preferred_element_type=jnp.float32)
    m = e2 + mm
    idx = jnp.argmin(m, axis=0)
    o_ref[0, 0] = idx.astype(jnp.int32)


def _tc_argmin(xr, weight, nb, off):
    B, d, HW = xr.shape
    K = weight.shape[1]
    out = pl.pallas_call(
        _tc_argmin_body,
        grid=(nb,),
        in_specs=[
            pl.BlockSpec((1, d, HW), lambda i: (i + off, 0, 0)),
            pl.BlockSpec((d, K), lambda i: (0, 0)),
        ],
        out_specs=pl.BlockSpec((1, 1, HW), lambda i: (i, 0, 0)),
        out_shape=jax.ShapeDtypeStruct((nb, 1, HW), jnp.int32),
    )(xr, weight)
    return out.reshape(nb, HW)


def _make_sc_gather(nb, d, K, HW):
    wpb = 32 // nb
    dr = d // wpb

    def body(w_hbm, idx_hbm, out_hbm, w_v, idx_v, out_v):
        info = plsc.get_sparse_core_info()
        nc = info.num_cores
        wid = lax.axis_index("s") * nc + lax.axis_index("c")
        b = wid // wpb
        d0 = (wid % wpb) * dr
        pltpu.sync_copy(idx_hbm.at[b], idx_v)
        pltpu.sync_copy(w_hbm.at[pl.ds(d0, dr)], w_v)

        @plsc.parallel_loop(0, HW // 16, unroll=8)
        def chunk(c):
            iv = idx_v[pl.ds(c * 16, 16)]
            for dl in range(dr):
                dv = jnp.full((16,), dl, jnp.int32)
                out_v[dl, pl.ds(c * 16, 16)] = plsc.load_gather(w_v, [dv, iv])
        pltpu.sync_copy(out_v, out_hbm.at[b, pl.ds(d0, dr)])

    return pl.kernel(
        body,
        out_type=jax.ShapeDtypeStruct((nb, d, HW), jnp.float32),
        mesh=plsc.VectorSubcoreMesh(core_axis_name="c", subcore_axis_name="s"),
        scratch_types=[
            pltpu.VMEM((dr, K), jnp.float32),
            pltpu.VMEM((HW,), jnp.int32),
            pltpu.VMEM((dr, HW), jnp.float32),
        ],
        compiler_params=pltpu.CompilerParams(
            use_tc_tiling_on_sc=False, needs_layout_passes=False),
    )


def kernel(x, weight):
    B, d, H, W = x.shape
    K = weight.shape[1]
    HW = H * W
    nchunk = 2
    nb = B // nchunk
    xr = x.reshape(B, d, HW)

    sc_gather = _make_sc_gather(nb, d, K, HW)
    qs, idxs = [], []
    for c in range(nchunk):
        idx_c = _tc_argmin(xr, weight, nb, c * nb)
        qs.append(sc_gather(weight, idx_c))
        idxs.append(idx_c)
    quant = jnp.concatenate([q.reshape(nb, d, H, W) for q in qs], axis=0)
    idx = jnp.concatenate(idxs, axis=0)

    return quant, idx.reshape(B, H, W)


# speedup vs baseline: 1.0134x; 1.0134x over previous
import functools

import jax
import jax.numpy as jnp
from jax import lax
from jax.experimental import pallas as pl
from jax.experimental.pallas import tpu as pltpu
from jax.experimental.pallas import tpu_sc as plsc


def _tc_argmin_body(x_ref, w_ref, o_ref):
    xb = x_ref[0]
    w = w_ref[...]
    e2 = jnp.sum(w * w, axis=0)[:, None]
    mm = lax.dot_general(w * -2.0, xb, (((0,), (0,)), ((), ())),
                         preferred_element_type=jnp.float32)
    m = e2 + mm
    idx = jnp.argmin(m, axis=0)
    o_ref[0, 0] = idx.astype(jnp.int32)


def _tc_argmin(xr, weight, nb, off):
    B, d, HW = xr.shape
    K = weight.shape[1]
    out = pl.pallas_call(
        _tc_argmin_body,
        grid=(nb,),
        in_specs=[
            pl.BlockSpec((1, d, HW), lambda i: (i + off, 0, 0)),
            pl.BlockSpec((d, K), lambda i: (0, 0)),
        ],
        out_specs=pl.BlockSpec((1, 1, HW), lambda i: (i, 0, 0)),
        out_shape=jax.ShapeDtypeStruct((nb, 1, HW), jnp.int32),
    )(xr, weight)
    return out.reshape(nb, HW)


def _make_sc_gather(nb, d, K, HW):
    wpb = 32 // nb
    dr = d // wpb

    def body(w_hbm, idx_hbm, out_hbm, w_v, idx_v, out_v):
        info = plsc.get_sparse_core_info()
        nc = info.num_cores
        wid = lax.axis_index("s") * nc + lax.axis_index("c")
        b = wid // wpb
        d0 = (wid % wpb) * dr
        pltpu.sync_copy(idx_hbm.at[b], idx_v)
        pltpu.sync_copy(w_hbm.at[pl.ds(d0, dr)], w_v)

        @plsc.parallel_loop(0, HW // 16, unroll=16)
        def chunk(c):
            iv = idx_v[pl.ds(c * 16, 16)]
            for dl in range(dr):
                dv = jnp.full((16,), dl, jnp.int32)
                out_v[dl, pl.ds(c * 16, 16)] = plsc.load_gather(w_v, [dv, iv])
        pltpu.sync_copy(out_v, out_hbm.at[b, pl.ds(d0, dr)])

    return pl.kernel(
        body,
        out_type=jax.ShapeDtypeStruct((nb, d, HW), jnp.float32),
        mesh=plsc.VectorSubcoreMesh(core_axis_name="c", subcore_axis_name="s"),
        scratch_types=[
            pltpu.VMEM((dr, K), jnp.float32),
            pltpu.VMEM((HW,), jnp.int32),
            pltpu.VMEM((dr, HW), jnp.float32),
        ],
        compiler_params=pltpu.CompilerParams(
            use_tc_tiling_on_sc=False, needs_layout_passes=False),
    )


def kernel(x, weight):
    B, d, H, W = x.shape
    K = weight.shape[1]
    HW = H * W
    nchunk = 2
    nb = B // nchunk
    xr = x.reshape(B, d, HW)

    sc_gather = _make_sc_gather(nb, d, K, HW)
    qs, idxs = [], []
    for c in range(nchunk):
        idx_c = _tc_argmin(xr, weight, nb, c * nb)
        qs.append(sc_gather(weight, idx_c))
        idxs.append(idx_c)
    quant = jnp.concatenate(qs, axis=0)
    idx = jnp.concatenate(idxs, axis=0)

    return quant.reshape(B, d, H, W), idx.reshape(B, H, W)


# speedup vs baseline: 1.0336x; 1.0200x over previous
import functools

import jax
import jax.numpy as jnp
from jax import lax
from jax.experimental import pallas as pl
from jax.experimental.pallas import tpu as pltpu
from jax.experimental.pallas import tpu_sc as plsc


def _tc_argmin_body(x_ref, w_ref, o_ref):
    xb = x_ref[0]
    w = w_ref[...]
    e2 = jnp.sum(w * w, axis=0)[:, None]
    mm = lax.dot_general(w * -2.0, xb, (((0,), (0,)), ((), ())),
                         preferred_element_type=jnp.float32)
    m = e2 + mm
    idx = jnp.argmin(m, axis=0)
    o_ref[0, 0] = idx.astype(jnp.int32)


def _tc_argmin(xr, weight, nb, off):
    B, d, HW = xr.shape
    K = weight.shape[1]
    out = pl.pallas_call(
        _tc_argmin_body,
        grid=(nb,),
        in_specs=[
            pl.BlockSpec((1, d, HW), lambda i: (i + off, 0, 0)),
            pl.BlockSpec((d, K), lambda i: (0, 0)),
        ],
        out_specs=pl.BlockSpec((1, 1, HW), lambda i: (i, 0, 0)),
        out_shape=jax.ShapeDtypeStruct((nb, 1, HW), jnp.int32),
    )(xr, weight)
    return out.reshape(nb, HW)


def _make_sc_gather(nb, d, K, HW):
    wpb = 32 // nb
    dr = d // wpb

    def body(w_hbm, idx_hbm, out_hbm, w_v, idx_v, out_v):
        info = plsc.get_sparse_core_info()
        nc = info.num_cores
        wid = lax.axis_index("s") * nc + lax.axis_index("c")
        b = wid // wpb
        d0 = (wid % wpb) * dr
        pltpu.sync_copy(idx_hbm.at[b], idx_v)
        pltpu.sync_copy(w_hbm.at[pl.ds(d0, dr)], w_v)

        @plsc.parallel_loop(0, HW // 16, unroll=8)
        def chunk(c):
            iv = idx_v[pl.ds(c * 16, 16)]
            for dl in range(dr):
                dv = jnp.full((16,), dl, jnp.int32)
                out_v[dl, pl.ds(c * 16, 16)] = plsc.load_gather(w_v, [dv, iv])
        pltpu.sync_copy(out_v, out_hbm.at[b, pl.ds(d0, dr)])

    return pl.kernel(
        body,
        out_type=jax.ShapeDtypeStruct((nb, d, HW), jnp.float32),
        mesh=plsc.VectorSubcoreMesh(core_axis_name="c", subcore_axis_name="s"),
        scratch_types=[
            pltpu.VMEM((dr, K), jnp.float32),
            pltpu.VMEM((HW,), jnp.int32),
            pltpu.VMEM((dr, HW), jnp.float32),
        ],
        compiler_params=pltpu.CompilerParams(
            use_tc_tiling_on_sc=False, needs_layout_passes=False),
    )


def kernel(x, weight):
    B, d, H, W = x.shape
    K = weight.shape[1]
    HW = H * W
    sizes = (B // 2, B // 4, B // 4)
    xr = x.reshape(B, d, HW)

    gathers = {nb: _make_sc_gather(nb, d, K, HW) for nb in set(sizes)}
    qs, idxs = [], []
    off = 0
    for nb in sizes:
        idx_c = _tc_argmin(xr, weight, nb, off)
        qs.append(gathers[nb](weight, idx_c))
        idxs.append(idx_c)
        off += nb
    quant = jnp.concatenate(qs, axis=0)
    idx = jnp.concatenate(idxs, axis=0)

    return quant.reshape(B, d, H, W), idx.reshape(B, H, W)


# speedup vs baseline: 1.0655x; 1.0308x over previous
import jax
import jax.numpy as jnp
from jax import lax
from jax.experimental import pallas as pl
from jax.experimental.pallas import tpu as pltpu
from jax.experimental.pallas import tpu_sc as plsc


def _tc_argmin_body(x_ref, w_ref, o_ref):
    xb = x_ref[0]
    w = w_ref[...]
    e2 = jnp.sum(w * w, axis=0)[:, None]
    mm = lax.dot_general(w * -2.0, xb, (((0,), (0,)), ((), ())),
                         preferred_element_type=jnp.float32)
    m = e2 + mm
    idx = jnp.argmin(m, axis=0)
    o_ref[0, 0] = idx.astype(jnp.int32)


def _tc_argmin(xr, weight, nb, off):
    B, d, HW = xr.shape
    K = weight.shape[1]
    out = pl.pallas_call(
        _tc_argmin_body,
        grid=(nb,),
        in_specs=[
            pl.BlockSpec((1, d, HW), lambda i: (i + off, 0, 0)),
            pl.BlockSpec((d, K), lambda i: (0, 0)),
        ],
        out_specs=pl.BlockSpec((1, 1, HW), lambda i: (i, 0, 0)),
        out_shape=jax.ShapeDtypeStruct((nb, 1, HW), jnp.int32),
    )(xr, weight)
    return out.reshape(nb, HW)


def _make_sc_gather(nb, d, K, HW):
    wpb = 32 // nb
    dr = d // wpb

    def body(w_hbm, idx_hbm, out_hbm, w_v, idx_v, out_v):
        info = plsc.get_sparse_core_info()
        nc = info.num_cores
        wid = lax.axis_index("s") * nc + lax.axis_index("c")
        b = wid // wpb
        d0 = (wid % wpb) * dr
        pltpu.sync_copy(idx_hbm.at[b], idx_v)
        pltpu.sync_copy(w_hbm.at[pl.ds(d0, dr)], w_v)

        @plsc.parallel_loop(0, HW // 16, unroll=8)
        def chunk(c):
            iv = idx_v[pl.ds(c * 16, 16)]
            for dl in range(dr):
                dv = jnp.full((16,), dl, jnp.int32)
                out_v[dl, pl.ds(c * 16, 16)] = plsc.load_gather(w_v, [dv, iv])
        pltpu.sync_copy(out_v, out_hbm.at[b, pl.ds(d0, dr)])

    return pl.kernel(
        body,
        out_type=jax.ShapeDtypeStruct((nb, d, HW), jnp.float32),
        mesh=plsc.VectorSubcoreMesh(core_axis_name="c", subcore_axis_name="s"),
        scratch_types=[
            pltpu.VMEM((dr, K), jnp.float32),
            pltpu.VMEM((HW,), jnp.int32),
            pltpu.VMEM((dr, HW), jnp.float32),
        ],
        compiler_params=pltpu.CompilerParams(
            use_tc_tiling_on_sc=False, needs_layout_passes=False),
    )


def kernel(x, weight):
    B, d, H, W = x.shape
    K = weight.shape[1]
    HW = H * W
    nchunk = 2
    nb = B // nchunk
    xr = x.reshape(B, d, HW)

    sc_gather = _make_sc_gather(nb, d, K, HW)
    qs, idxs = [], []
    for c in range(nchunk):
        idx_c = _tc_argmin(xr, weight, nb, c * nb)
        qs.append(sc_gather(weight, idx_c))
        idxs.append(idx_c)
    quant = jnp.concatenate(qs, axis=0)
    idx = jnp.concatenate(idxs, axis=0)

    return quant.reshape(B, d, H, W), idx.reshape(B, H, W)
